# baseline (speedup 1.0000x reference)
import functools

import jax
import jax.numpy as jnp
from jax import lax
from jax.experimental import pallas as pl
from jax.experimental.pallas import tpu as pltpu
from jax.experimental.pallas import tpu_sc as plsc

NUM_EMB = 1000000
D = 64
B = 4096
L = 50

NC = 2
NS = 16
NW = NC * NS
BPW = B // NW
C = 8
NCHUNK = BPW // C
NG = D // 16


def _emb_bag_kernel(x_hbm, w_hbm, out_hbm, idx_v, rows_v, out_v, sem):
    wid = lax.axis_index("s") * NC + lax.axis_index("c")
    base = wid * BPW

    @pl.loop(0, NCHUNK)
    def _chunk(t):
        bag = base + t * C
        pltpu.sync_copy(x_hbm.at[pl.ds(bag, C)], idx_v)
        copies = [
            pltpu.async_copy(w_hbm.at[idx_v.at[c]], rows_v.at[c], sem)
            for c in range(C)
        ]
        for cp in copies:
            cp.wait()
        for c in range(C):
            def body(j, accs):
                return tuple(
                    accs[g] + rows_v[c, j, pl.ds(g * 16, 16)]
                    for g in range(NG)
                )
            accs = lax.fori_loop(
                0, L, body,
                tuple(jnp.zeros((16,), jnp.float32) for _ in range(NG)),
            )
            for g in range(NG):
                out_v[c, pl.ds(g * 16, 16)] = accs[g] * (1.0 / L)
        pltpu.sync_copy(out_v, out_hbm.at[pl.ds(bag, C)])


@jax.jit
def _emb_bag(x, weight):
    mesh = plsc.VectorSubcoreMesh(core_axis_name="c", subcore_axis_name="s")
    f = pl.kernel(
        _emb_bag_kernel,
        out_type=jax.ShapeDtypeStruct((B, D), jnp.float32),
        mesh=mesh,
        scratch_types=[
            pltpu.VMEM((C, L), jnp.int32),
            pltpu.VMEM((C, L, D), jnp.float32),
            pltpu.VMEM((C, D), jnp.float32),
            pltpu.SemaphoreType.DMA,
        ],
        compiler_params=pltpu.CompilerParams(use_tc_tiling_on_sc=False),
    )
    return f(x, weight)


def kernel(x, weight):
    return _emb_bag(x.astype(jnp.int32), weight)


# speedup vs baseline: 1.5344x; 1.5344x over previous
import functools

import jax
import jax.numpy as jnp
from jax import lax
from jax.experimental import pallas as pl
from jax.experimental.pallas import tpu as pltpu
from jax.experimental.pallas import tpu_sc as plsc

NUM_EMB = 1000000
D = 64
B = 4096
L = 50

NC = 2
NS = 16
NW = NC * NS
BPW = B // NW
C = 8
CL = C * L
NCHUNK = BPW // C
NG = D // 16
NBLK = (CL + 15) // 16


def _emb_bag_kernel(xf_hbm, w_hbm, out_hbm, idx_v, rows_v, out_v, sem):
    wid = lax.axis_index("s") * NC + lax.axis_index("c")
    base = wid * BPW

    @pl.loop(0, NCHUNK)
    def _chunk(t):
        bag = base + t * C
        pltpu.sync_copy(xf_hbm.at[pl.ds(bag * L, CL)], idx_v)
        @pl.loop(0, NBLK)
        def _blk(mb):
            v = idx_v[pl.ds(mb * 16, 16)]
            for l in range(16):
                r = v[l]
                pltpu.async_copy(
                    w_hbm.at[pl.ds(r, 1), :],
                    rows_v.at[pl.ds(mb * 16 + l, 1), :],
                    sem,
                )
        pltpu.make_async_copy(
            w_hbm.at[pl.ds(0, CL), :], rows_v, sem
        ).wait()
        for c in range(C):
            def body(j, accs):
                return tuple(
                    accs[g] + rows_v[c * L + j, pl.ds(g * 16, 16)]
                    for g in range(NG)
                )
            accs = lax.fori_loop(
                0, L, body,
                tuple(jnp.zeros((16,), jnp.float32) for _ in range(NG)),
            )
            for g in range(NG):
                out_v[c, pl.ds(g * 16, 16)] = accs[g] * (1.0 / L)
        pltpu.sync_copy(out_v, out_hbm.at[pl.ds(bag, C)])


@jax.jit
def _emb_bag(x, weight):
    xf = x.reshape(B * L)
    mesh = plsc.VectorSubcoreMesh(core_axis_name="c", subcore_axis_name="s")
    f = pl.kernel(
        _emb_bag_kernel,
        out_type=jax.ShapeDtypeStruct((B, D), jnp.float32),
        mesh=mesh,
        scratch_types=[
            pltpu.VMEM((CL,), jnp.int32),
            pltpu.VMEM((CL, D), jnp.float32),
            pltpu.VMEM((C, D), jnp.float32),
            pltpu.SemaphoreType.DMA,
        ],
    )
    return f(xf, weight)


def kernel(x, weight):
    return _emb_bag(x.astype(jnp.int32), weight)
